# Initial kernel scaffold; baseline (speedup 1.0000x reference)
import jax
import jax.numpy as jnp
from jax.experimental import pallas as pl


def kernel(inputs, ind1, w1, lambda1):
    raise NotImplementedError("write your pallas kernel here")


# speedup vs baseline: 5.6954x; 5.6954x over previous
import functools

import jax
import jax.numpy as jnp
from jax.experimental import pallas as pl
from jax.experimental.pallas import tpu as pltpu


def _build_at_kernel(idx_ref, w_ref, lam_ref, at_ref):
    h = at_ref.shape[0]
    k_fan = idx_ref.shape[0]
    row = jax.lax.broadcasted_iota(jnp.int32, (h, h), 0)
    mt = jnp.zeros((h, h), dtype=jnp.float32)
    for k in range(k_fan):
        hit = (row == idx_ref[k:k + 1, :]).astype(jnp.float32)
        mt = mt + w_ref[0, k] * hit
    mt5 = mt
    for _ in range(4):
        mt5 = jnp.dot(mt, mt5, preferred_element_type=jnp.float32)
    lam = lam_ref[0, 0]
    scale = lam * lam * lam * lam * lam
    at_ref[...] = scale * mt5


def _apply_kernel(at_ref, x_ref, o_ref, *, h, b):
    at = at_ref[...]
    for i in range(b):
        xb = x_ref[i * h:(i + 1) * h, :]
        ob = jax.lax.dot_general(
            at, xb, (((0,), (0,)), ((), ())),
            preferred_element_type=jnp.float32)
        o_ref[i * h:(i + 1) * h, :] = ob


def kernel(inputs, ind1, w1, lambda1):
    b, h, w, c = inputs.shape
    k_fan = ind1.shape[0]

    idx = ind1[..., 0].astype(jnp.int32)
    wv = w1.reshape(1, k_fan).astype(jnp.float32)
    lam = lambda1.reshape(1, 1).astype(jnp.float32)

    at = pl.pallas_call(
        _build_at_kernel,
        out_shape=jax.ShapeDtypeStruct((h, h), jnp.float32),
        in_specs=[
            pl.BlockSpec(memory_space=pltpu.VMEM),
            pl.BlockSpec(memory_space=pltpu.SMEM),
            pl.BlockSpec(memory_space=pltpu.SMEM),
        ],
        out_specs=pl.BlockSpec(memory_space=pltpu.VMEM),
    )(idx, wv, lam)

    cols = w * c
    blk = 4096 if cols % 4096 == 0 else cols
    nb = cols // blk
    x2 = inputs.reshape(b * h, cols)

    body = functools.partial(_apply_kernel, h=h, b=b)
    out2 = pl.pallas_call(
        body,
        grid=(nb,),
        in_specs=[
            pl.BlockSpec((h, h), lambda j: (0, 0)),
            pl.BlockSpec((b * h, blk), lambda j: (0, j)),
        ],
        out_specs=pl.BlockSpec((b * h, blk), lambda j: (0, j)),
        out_shape=jax.ShapeDtypeStruct((b * h, cols), jnp.float32),
    )(at, x2)

    return out2.reshape(b, h, w, c)


# speedup vs baseline: 14.9835x; 2.6308x over previous
import functools

import jax
import jax.numpy as jnp
from jax.experimental import pallas as pl
from jax.experimental.pallas import tpu as pltpu


def _build_at_kernel(idx_ref, w_ref, lam_ref, at_ref):
    h = at_ref.shape[0]
    k_fan = idx_ref.shape[0]
    row = jax.lax.broadcasted_iota(jnp.int32, (h, h), 0)
    mt = jnp.zeros((h, h), dtype=jnp.float32)
    for k in range(k_fan):
        hit = (row == idx_ref[k:k + 1, :]).astype(jnp.float32)
        mt = mt + w_ref[0, k] * hit
    mt5 = mt
    for _ in range(4):
        mt5 = jnp.dot(mt, mt5, preferred_element_type=jnp.float32)
    lam = lam_ref[0, 0]
    scale = lam * lam * lam * lam * lam
    at_ref[...] = scale * mt5


def _apply_kernel(at_ref, x_ref, o_ref):
    h, wblk, c = x_ref.shape
    x2 = x_ref[...].reshape(h, wblk * c)
    ob = jax.lax.dot_general(
        at_ref[...], x2, (((0,), (0,)), ((), ())),
        preferred_element_type=jnp.float32)
    o_ref[...] = ob.reshape(h, wblk, c)


def kernel(inputs, ind1, w1, lambda1):
    b, h, w, c = inputs.shape
    k_fan = ind1.shape[0]

    idx = ind1[..., 0].astype(jnp.int32)
    wv = w1.reshape(1, k_fan).astype(jnp.float32)
    lam = lambda1.reshape(1, 1).astype(jnp.float32)

    at = pl.pallas_call(
        _build_at_kernel,
        out_shape=jax.ShapeDtypeStruct((h, h), jnp.float32),
        in_specs=[
            pl.BlockSpec(memory_space=pltpu.VMEM),
            pl.BlockSpec(memory_space=pltpu.SMEM),
            pl.BlockSpec(memory_space=pltpu.SMEM),
        ],
        out_specs=pl.BlockSpec(memory_space=pltpu.VMEM),
    )(idx, wv, lam)

    wblk = 64
    x3 = inputs.reshape(b * h, w, c)
    out3 = pl.pallas_call(
        _apply_kernel,
        grid=(b, w // wblk),
        in_specs=[
            pl.BlockSpec((h, h), lambda i, j: (0, 0)),
            pl.BlockSpec((h, wblk, c), lambda i, j: (i, j, 0)),
        ],
        out_specs=pl.BlockSpec((h, wblk, c), lambda i, j: (i, j, 0)),
        out_shape=jax.ShapeDtypeStruct((b * h, w, c), jnp.float32),
    )(at, x3)

    return out3.reshape(b, h, w, c)


# speedup vs baseline: 19.7735x; 1.3197x over previous
import jax
import jax.numpy as jnp
from jax.experimental import pallas as pl
from jax.experimental.pallas import tpu as pltpu


def _fused_kernel(idx_ref, w_ref, lam_ref, x_ref, o_ref, at_ref):
    i = pl.program_id(0)
    j = pl.program_id(1)

    @pl.when(jnp.logical_and(i == 0, j == 0))
    def _build():
        h = at_ref.shape[0]
        k_fan = idx_ref.shape[0]
        row = jax.lax.broadcasted_iota(jnp.int32, (h, h), 0)
        mt = jnp.zeros((h, h), dtype=jnp.float32)
        for k in range(k_fan):
            hit = (row == idx_ref[k:k + 1, :]).astype(jnp.float32)
            mt = mt + w_ref[0, k] * hit
        mt5 = mt
        for _ in range(4):
            mt5 = jnp.dot(mt, mt5, preferred_element_type=jnp.float32)
        lam = lam_ref[0, 0]
        at_ref[...] = (lam * lam * lam * lam * lam) * mt5

    h, wblk, c = x_ref.shape
    x2 = x_ref[...].reshape(h, wblk * c)
    ob = jax.lax.dot_general(
        at_ref[...], x2, (((0,), (0,)), ((), ())),
        preferred_element_type=jnp.float32)
    o_ref[...] = ob.reshape(h, wblk, c)


def kernel(inputs, ind1, w1, lambda1):
    b, h, w, c = inputs.shape
    k_fan = ind1.shape[0]

    idx = ind1[..., 0].astype(jnp.int32)
    wv = w1.reshape(1, k_fan).astype(jnp.float32)
    lam = lambda1.reshape(1, 1).astype(jnp.float32)

    wblk = 128
    x3 = inputs.reshape(b * h, w, c)
    out3 = pl.pallas_call(
        _fused_kernel,
        grid=(b, w // wblk),
        in_specs=[
            pl.BlockSpec(memory_space=pltpu.VMEM),
            pl.BlockSpec(memory_space=pltpu.SMEM),
            pl.BlockSpec(memory_space=pltpu.SMEM),
            pl.BlockSpec((h, wblk, c), lambda i, j: (i, j, 0)),
        ],
        out_specs=pl.BlockSpec((h, wblk, c), lambda i, j: (i, j, 0)),
        out_shape=jax.ShapeDtypeStruct((b * h, w, c), jnp.float32),
        scratch_shapes=[pltpu.VMEM((h, h), jnp.float32)],
    )(idx, wv, lam, x3)

    return out3.reshape(b, h, w, c)


# speedup vs baseline: 21.4490x; 1.0847x over previous
import jax
import jax.numpy as jnp
from jax.experimental import pallas as pl
from jax.experimental.pallas import tpu as pltpu


def _fused_kernel(idx_ref, w_ref, lam_ref, x_ref, o_ref, at_ref):
    i = pl.program_id(0)
    j = pl.program_id(1)

    @pl.when(jnp.logical_and(i == 0, j == 0))
    def _build():
        h = at_ref.shape[0]
        k_fan = idx_ref.shape[0]
        row = jax.lax.broadcasted_iota(jnp.int32, (h, h), 0)
        mt = jnp.zeros((h, h), dtype=jnp.float32)
        for k in range(k_fan):
            hit = (row == idx_ref[k:k + 1, :]).astype(jnp.float32)
            mt = mt + w_ref[0, k] * hit
        mt5 = mt
        for _ in range(4):
            mt5 = jnp.dot(mt, mt5, preferred_element_type=jnp.float32)
        lam = lam_ref[0, 0]
        at_ref[...] = (lam * lam * lam * lam * lam) * mt5

    h, wblk, c = x_ref.shape
    x2 = x_ref[...].reshape(h, wblk * c)
    ob = jax.lax.dot_general(
        at_ref[...], x2, (((0,), (0,)), ((), ())),
        preferred_element_type=jnp.float32)
    o_ref[...] = ob.reshape(h, wblk, c)


def kernel(inputs, ind1, w1, lambda1):
    b, h, w, c = inputs.shape
    k_fan = ind1.shape[0]

    idx = ind1[..., 0].astype(jnp.int32)
    wv = w1.reshape(1, k_fan).astype(jnp.float32)
    lam = lambda1.reshape(1, 1).astype(jnp.float32)

    wblk = 256
    x3 = inputs.reshape(b * h, w, c)
    out3 = pl.pallas_call(
        _fused_kernel,
        grid=(b, w // wblk),
        in_specs=[
            pl.BlockSpec(memory_space=pltpu.VMEM),
            pl.BlockSpec(memory_space=pltpu.SMEM),
            pl.BlockSpec(memory_space=pltpu.SMEM),
            pl.BlockSpec((h, wblk, c), lambda i, j: (i, j, 0)),
        ],
        out_specs=pl.BlockSpec((h, wblk, c), lambda i, j: (i, j, 0)),
        out_shape=jax.ShapeDtypeStruct((b * h, w, c), jnp.float32),
        scratch_shapes=[pltpu.VMEM((h, h), jnp.float32)],
    )(idx, wv, lam, x3)

    return out3.reshape(b, h, w, c)
